# baseline (speedup 1.0000x reference)
import functools

import jax
import jax.numpy as jnp
from jax import lax
from jax.experimental import pallas as pl
from jax.experimental.pallas import tpu as pltpu
from jax.experimental.pallas import tpu_sc as plsc

_EPS = 1e-8
_NBITS = 8
_QSCALE = 2.0 ** (_NBITS - 1)

_B, _H, _W, _C = 4, 224, 224, 192
_NW = 32
_WPB = _NW // _B
_RPW = _H // _WPB
_HW = _W // 2
_NCH = _RPW * 2


def _sc_body(x_hbm, p_hbm, o_hbm, ib0, ib1, ob0, ob1, pb,
             si0, si1, so0, so1):
    wid = lax.axis_index("s") * 2 + lax.axis_index("c")
    bidx = wid // _WPB
    h0 = (wid % _WPB) * _RPW

    pltpu.sync_copy(p_hbm, pb)
    pv = pb[pl.ds(0, 16)]
    a = pv[0]
    b = pv[1]
    c = pv[2]
    d = pv[3]

    ibufs = (ib0, ib1)
    obufs = (ob0, ob1)
    isems = (si0, si1)
    osems = (so0, so1)

    def src(t):
        return x_hbm.at[bidx, h0 + t // 2, pl.ds((t % 2) * _HW, _HW)]

    def dst(t):
        return o_hbm.at[bidx, h0 + t // 2, pl.ds((t % 2) * _HW, _HW)]

    def compute(ib, ob):
        def row(j, carry):
            for k in range(_C // 16):
                v = ib[j, pl.ds(k * 16, 16)]
                y = v * a + b
                y = jnp.minimum(jnp.maximum(y, 0.0), 63.0)
                q = y.astype(jnp.int32).astype(jnp.float32)
                ob[j, pl.ds(k * 16, 16)] = q * c + d
            return carry
        lax.fori_loop(0, _HW, row, 0)

    pltpu.async_copy(src(0), ibufs[0], isems[0])
    pltpu.async_copy(src(1), ibufs[1], isems[1])

    def step(g, carry):
        for p in range(2):
            t = 2 * g + p
            pltpu.make_async_copy(src(t), ibufs[p], isems[p]).wait()

            @pl.when(g >= 1)
            def _():
                pltpu.make_async_copy(obufs[p], dst(t - 2), osems[p]).wait()

            compute(ibufs[p], obufs[p])
            pltpu.async_copy(obufs[p], dst(t), osems[p])

            @pl.when(g + 1 < _NCH // 2)
            def _():
                pltpu.async_copy(src(t + 2), ibufs[p], isems[p])
        return carry

    lax.fori_loop(0, _NCH // 2, step, 0)

    pltpu.make_async_copy(obufs[0], dst(_NCH - 2), osems[0]).wait()
    pltpu.make_async_copy(obufs[1], dst(_NCH - 1), osems[1]).wait()


def kernel(input_data, lut_values, threshold):
    thr = jnp.asarray(threshold, jnp.float32)
    lut0 = lut_values[0]
    step = lut_values[1] - lut_values[0]
    a = _QSCALE / ((thr + _EPS) * step)
    b = 0.5 - lut0 / step
    c = step * thr / _QSCALE
    d = lut0 * thr / _QSCALE
    params = jnp.zeros((16,), jnp.float32).at[0].set(a).at[1].set(b) \
        .at[2].set(c).at[3].set(d)

    mesh = plsc.VectorSubcoreMesh(core_axis_name="c", subcore_axis_name="s")
    run = functools.partial(
        pl.kernel,
        mesh=mesh,
        out_type=jax.ShapeDtypeStruct((_B, _H, _W, _C), jnp.float32),
        scratch_types=[
            pltpu.VMEM((_HW, _C), jnp.float32),
            pltpu.VMEM((_HW, _C), jnp.float32),
            pltpu.VMEM((_HW, _C), jnp.float32),
            pltpu.VMEM((_HW, _C), jnp.float32),
            pltpu.VMEM((16,), jnp.float32),
            pltpu.SemaphoreType.DMA,
            pltpu.SemaphoreType.DMA,
            pltpu.SemaphoreType.DMA,
            pltpu.SemaphoreType.DMA,
        ],
        compiler_params=pltpu.CompilerParams(use_tc_tiling_on_sc=True),
    )(_sc_body)
    return run(input_data, params)


# speedup vs baseline: 1.0010x; 1.0010x over previous
import functools

import jax
import jax.numpy as jnp
from jax import lax
from jax.experimental import pallas as pl
from jax.experimental.pallas import tpu as pltpu
from jax.experimental.pallas import tpu_sc as plsc

_EPS = 1e-8
_NBITS = 8
_QSCALE = 2.0 ** (_NBITS - 1)

_B, _H, _W, _C = 4, 224, 224, 192
_NW = 32
_WPB = _NW // _B
_RPW = _H // _WPB
_HW = _W // 2
_NCH = _RPW * 2


def _sc_body(x_hbm, p_hbm, o_hbm, ib0, ib1, ob0, ob1, pb,
             si0, si1, so0, so1):
    wid = lax.axis_index("s") * 2 + lax.axis_index("c")
    bidx = wid // _WPB
    h0 = (wid % _WPB) * _RPW

    pltpu.sync_copy(p_hbm, pb)
    pv = pb[pl.ds(0, 16)]
    a = pv[0]
    b = pv[1]
    c = pv[2]
    d = pv[3]

    ibufs = (ib0, ib1)
    obufs = (ob0, ob1)
    isems = (si0, si1)
    osems = (so0, so1)

    def src(t):
        return x_hbm.at[bidx, h0 + t // 2, pl.ds((t % 2) * _HW, _HW)]

    def dst(t):
        return o_hbm.at[bidx, h0 + t // 2, pl.ds((t % 2) * _HW, _HW)]

    def compute(ib, ob):
        def row(j, carry):
            for k in range(_C // 16):
                v = ib[j, pl.ds(k * 16, 16)]
                y = v * a + b
                y = jnp.minimum(jnp.maximum(y, 0.0), 63.0)
                q = y.astype(jnp.int32).astype(jnp.float32)
                ob[j, pl.ds(k * 16, 16)] = q * c + d
            return carry
        lax.fori_loop(0, _HW, row, 0)

    pltpu.async_copy(src(0), ibufs[0], isems[0])
    pltpu.async_copy(src(1), ibufs[1], isems[1])

    def step(g, carry):
        for p in range(2):
            t = 2 * g + p
            pltpu.make_async_copy(src(t), ibufs[p], isems[p]).wait()

            @pl.when(g >= 1)
            def _():
                pltpu.make_async_copy(obufs[p], dst(t - 2), osems[p]).wait()

            compute(ibufs[p], obufs[p])
            pltpu.async_copy(obufs[p], dst(t), osems[p])

            @pl.when(g + 1 < _NCH // 2)
            def _():
                pltpu.async_copy(src(t + 2), ibufs[p], isems[p])
        return carry

    lax.fori_loop(0, _NCH // 2, step, 0)

    pltpu.make_async_copy(obufs[0], dst(_NCH - 2), osems[0]).wait()
    pltpu.make_async_copy(obufs[1], dst(_NCH - 1), osems[1]).wait()


def kernel(input_data, lut_values, threshold):
    thr = jnp.asarray(threshold, jnp.float32)
    lut0 = lut_values[0]
    step = lut_values[1] - lut_values[0]
    a = _QSCALE / ((thr + _EPS) * step)
    b = 0.5 - lut0 / step
    c = step * thr / _QSCALE
    d = lut0 * thr / _QSCALE
    params = jnp.zeros((16,), jnp.float32).at[0].set(a).at[1].set(b) \
        .at[2].set(c).at[3].set(d)

    mesh = plsc.VectorSubcoreMesh(core_axis_name="c", subcore_axis_name="s")
    run = functools.partial(
        pl.kernel,
        mesh=mesh,
        out_type=jax.ShapeDtypeStruct((_B, _H, _W, _C), jnp.float32),
        scratch_types=[
            pltpu.VMEM((_HW, _C), jnp.float32),
            pltpu.VMEM((_HW, _C), jnp.float32),
            pltpu.VMEM((_HW, _C), jnp.float32),
            pltpu.VMEM((_HW, _C), jnp.float32),
            pltpu.VMEM((16,), jnp.float32),
            pltpu.SemaphoreType.DMA,
            pltpu.SemaphoreType.DMA,
            pltpu.SemaphoreType.DMA,
            pltpu.SemaphoreType.DMA,
        ],
        compiler_params=pltpu.CompilerParams(
            use_tc_tiling_on_sc=True, needs_layout_passes=True),
    )(_sc_body)
    return run(input_data, params)


# speedup vs baseline: 1.0832x; 1.0821x over previous
import functools

import jax
import jax.numpy as jnp
from jax import lax
from jax.experimental import pallas as pl
from jax.experimental.pallas import tpu as pltpu
from jax.experimental.pallas import tpu_sc as plsc

_EPS = 1e-8
_NBITS = 8
_QSCALE = 2.0 ** (_NBITS - 1)

_B, _H, _W, _C = 4, 224, 224, 192
_NW = 32
_WPB = _NW // _B
_RPW = _H // _WPB
_HW = _W // 2
_NCH = _RPW * 2


def _sc_body(x_hbm, p_hbm, o_hbm, ib0, ib1, ob0, ob1, pb,
             si0, si1, so0, so1):
    wid = lax.axis_index("s") * 2 + lax.axis_index("c")
    bidx = wid // _WPB
    h0 = (wid % _WPB) * _RPW

    pltpu.sync_copy(p_hbm, pb)
    pv = pb[pl.ds(0, 16)]
    a = pv[0]
    b = pv[1]
    c = pv[2]
    d = pv[3]

    ibufs = (ib0, ib1)
    obufs = (ob0, ob1)
    isems = (si0, si1)
    osems = (so0, so1)

    def src(t):
        return x_hbm.at[bidx, h0 + t // 2, pl.ds((t % 2) * _HW, _HW)]

    def dst(t):
        return o_hbm.at[bidx, h0 + t // 2, pl.ds((t % 2) * _HW, _HW)]

    def compute(ib, ob):
        def row(j, carry):
            for k in range(_C // 16):
                v = ib[j, pl.ds(k * 16, 16)]
                y = v * a + b
                y = jnp.minimum(jnp.maximum(y, 0.0), 63.0)
                q = y.astype(jnp.int32).astype(jnp.float32)
                ob[j, pl.ds(k * 16, 16)] = q * c + d
            return carry
        lax.fori_loop(0, _HW, row, 0)

    pltpu.async_copy(src(0), ibufs[0], isems[0])
    pltpu.async_copy(src(1), ibufs[1], isems[1])

    def step(g, carry):
        for p in range(2):
            t = 2 * g + p
            pltpu.make_async_copy(src(t), ibufs[p], isems[p]).wait()

            @pl.when(g >= 1)
            def _():
                pltpu.make_async_copy(obufs[p], dst(t - 2), osems[p]).wait()

            compute(ibufs[p], obufs[p])
            pltpu.async_copy(obufs[p], dst(t), osems[p])

            @pl.when(g + 1 < _NCH // 2)
            def _():
                pltpu.async_copy(src(t + 2), ibufs[p], isems[p])
        return carry

    lax.fori_loop(0, _NCH // 2, step, 0)

    pltpu.make_async_copy(obufs[0], dst(_NCH - 2), osems[0]).wait()
    pltpu.make_async_copy(obufs[1], dst(_NCH - 1), osems[1]).wait()


_SH = 8
_K = 8
_NSLAB = _B * _H // _SH


def _tc_body(thr_ref, lut_ref, x_any, o_any, ibuf, obuf, isem, osem):
    thr = thr_ref[0]
    lut0 = lut_ref[0]
    step = lut_ref[1] - lut_ref[0]
    a = _QSCALE / ((thr + _EPS) * step)
    b = 0.5 - lut0 / step
    c = step * thr / _QSCALE
    d = lut0 * thr / _QSCALE

    spb = _H // _SH

    def src(s):
        return x_any.at[s // spb, pl.ds((s % spb) * _SH, _SH)]

    def dst(s):
        return o_any.at[s // spb, pl.ds((s % spb) * _SH, _SH)]

    for k in range(_K):
        pltpu.make_async_copy(src(k), ibuf.at[k], isem.at[k]).start()

    def group(g, carry):
        for k in range(_K):
            s = g * _K + k
            pltpu.make_async_copy(src(s), ibuf.at[k], isem.at[k]).wait()

            @pl.when(g >= 1)
            def _():
                pltpu.make_async_copy(obuf.at[k], dst(s - _K),
                                      osem.at[k]).wait()

            v = ibuf[k]
            y = v * a + b
            y = jnp.minimum(jnp.maximum(y, 0.0), 63.0)
            q = y.astype(jnp.int32).astype(jnp.float32)
            obuf[k] = q * c + d

            pltpu.make_async_copy(obuf.at[k], dst(s), osem.at[k]).start()

            @pl.when(s + _K < _NSLAB)
            def _():
                pltpu.make_async_copy(src(s + _K), ibuf.at[k],
                                      isem.at[k]).start()
        return carry

    lax.fori_loop(0, _NSLAB // _K, group, 0)

    for k in range(_K):
        pltpu.make_async_copy(obuf.at[k], dst(_NSLAB - _K + k),
                              osem.at[k]).wait()


def _tc_kernel(input_data, lut_values, threshold):
    thr = jnp.asarray(threshold, jnp.float32).reshape(1)
    return pl.pallas_call(
        _tc_body,
        in_specs=[
            pl.BlockSpec(memory_space=pltpu.SMEM),
            pl.BlockSpec(memory_space=pltpu.SMEM),
            pl.BlockSpec(memory_space=pl.ANY),
        ],
        out_specs=pl.BlockSpec(memory_space=pl.ANY),
        out_shape=jax.ShapeDtypeStruct((_B, _H, _W, _C), jnp.float32),
        scratch_shapes=[
            pltpu.VMEM((_K, _SH, _W, _C), jnp.float32),
            pltpu.VMEM((_K, _SH, _W, _C), jnp.float32),
            pltpu.SemaphoreType.DMA((_K,)),
            pltpu.SemaphoreType.DMA((_K,)),
        ],
    )(thr, lut_values, input_data)


def kernel(input_data, lut_values, threshold):
    return _tc_kernel(input_data, lut_values, threshold)


def _sc_kernel(input_data, lut_values, threshold):
    thr = jnp.asarray(threshold, jnp.float32)
    lut0 = lut_values[0]
    step = lut_values[1] - lut_values[0]
    a = _QSCALE / ((thr + _EPS) * step)
    b = 0.5 - lut0 / step
    c = step * thr / _QSCALE
    d = lut0 * thr / _QSCALE
    params = jnp.zeros((16,), jnp.float32).at[0].set(a).at[1].set(b) \
        .at[2].set(c).at[3].set(d)

    mesh = plsc.VectorSubcoreMesh(core_axis_name="c", subcore_axis_name="s")
    run = functools.partial(
        pl.kernel,
        mesh=mesh,
        out_type=jax.ShapeDtypeStruct((_B, _H, _W, _C), jnp.float32),
        scratch_types=[
            pltpu.VMEM((_HW, _C), jnp.float32),
            pltpu.VMEM((_HW, _C), jnp.float32),
            pltpu.VMEM((_HW, _C), jnp.float32),
            pltpu.VMEM((_HW, _C), jnp.float32),
            pltpu.VMEM((16,), jnp.float32),
            pltpu.SemaphoreType.DMA,
            pltpu.SemaphoreType.DMA,
            pltpu.SemaphoreType.DMA,
            pltpu.SemaphoreType.DMA,
        ],
        compiler_params=pltpu.CompilerParams(
            use_tc_tiling_on_sc=True, needs_layout_passes=True),
    )(_sc_body)
    return run(input_data, params)
